# Initial kernel scaffold; baseline (speedup 1.0000x reference)
import jax
import jax.numpy as jnp
from jax.experimental import pallas as pl


def kernel(x, Wg, bg, We, be):
    raise NotImplementedError("write your pallas kernel here")


# speedup vs baseline: 1.3795x; 1.3795x over previous
import functools

import jax
import jax.numpy as jnp
from jax.experimental import pallas as pl
from jax.experimental.pallas import tpu as pltpu


def _moe_kernel(x_ref, wg_ref, bg_row_ref, bg_col_ref, we_ref, be_ref,
                out_ref, logits_ref, w_ref, idx_ref, mask_ref,
                wscr, iscr):
    e = pl.program_id(1)
    num_e = pl.num_programs(1)
    x = x_ref[...]

    @pl.when(e == 0)
    def _router():
        wg = wg_ref[...]
        logits = jax.lax.dot_general(
            x, wg, (((1,), (1,)), ((), ())),
            preferred_element_type=jnp.float32) + bg_row_ref[...]
        logits_ref[...] = logits
        mx = jnp.max(logits, axis=1, keepdims=True)
        ex = jnp.exp(logits - mx)
        probs = ex / jnp.sum(ex, axis=1, keepdims=True)
        iota_e = jax.lax.broadcasted_iota(jnp.int32, probs.shape, 1)
        p0 = jnp.max(probs, axis=1, keepdims=True)
        i0 = jnp.min(jnp.where(probs == p0, iota_e, num_e),
                     axis=1, keepdims=True)
        probs1 = jnp.where(iota_e == i0, -1.0, probs)
        p1 = jnp.max(probs1, axis=1, keepdims=True)
        i1 = jnp.min(jnp.where(probs1 == p1, iota_e, num_e),
                     axis=1, keepdims=True)
        s = p0 + p1
        w0 = p0 / s
        w1 = p1 / s
        w_ref[:, 0:1] = w0
        w_ref[:, 1:2] = w1
        idx_ref[:, 0:1] = i0
        idx_ref[:, 1:2] = i1
        wscr[:, 0:1] = w0
        wscr[:, 1:2] = w1
        iscr[:, 0:1] = i0
        iscr[:, 1:2] = i1
        logits_t = jax.lax.dot_general(
            wg, x, (((1,), (1,)), ((), ())),
            preferred_element_type=jnp.float32) + bg_col_ref[...]
        mx_t = jnp.max(logits_t, axis=0, keepdims=True)
        ex_t = jnp.exp(logits_t - mx_t)
        probs_t = ex_t / jnp.sum(ex_t, axis=0, keepdims=True)
        iota_t = jax.lax.broadcasted_iota(jnp.int32, probs_t.shape, 0)
        p0_t = jnp.max(probs_t, axis=0, keepdims=True)
        i0_t = jnp.min(jnp.where(probs_t == p0_t, iota_t, num_e),
                       axis=0, keepdims=True)
        probs1_t = jnp.where(iota_t == i0_t, -1.0, probs_t)
        p1_t = jnp.max(probs1_t, axis=0, keepdims=True)
        i1_t = jnp.min(jnp.where(probs1_t == p1_t, iota_t, num_e),
                       axis=0, keepdims=True)
        mask_ref[:, 0, :] = (iota_t == i0_t).astype(jnp.int32)
        mask_ref[:, 1, :] = (iota_t == i1_t).astype(jnp.int32)

    w0 = wscr[:, 0:1]
    w1 = wscr[:, 1:2]
    i0 = iscr[:, 0:1]
    i1 = iscr[:, 1:2]
    we = we_ref[0]
    eo = jax.lax.dot_general(
        x, we, (((1,), (1,)), ((), ())),
        preferred_element_type=jnp.float32) + be_ref[0]
    w_e = jnp.where(i0 == e, w0, 0.0) + jnp.where(i1 == e, w1, 0.0)
    contrib = eo * w_e

    @pl.when(e == 0)
    def _init():
        out_ref[...] = contrib

    @pl.when(e > 0)
    def _acc():
        out_ref[...] = out_ref[...] + contrib


def kernel(x, Wg, bg, We, be):
    b, s, d = x.shape
    n = b * s
    num_e = Wg.shape[0]
    h = x.reshape(n, d)
    blk = 1024 if n % 1024 == 0 else n
    grid = (n // blk, num_e)

    out_shapes = (
        jax.ShapeDtypeStruct((n, d), jnp.float32),
        jax.ShapeDtypeStruct((n, num_e), jnp.float32),
        jax.ShapeDtypeStruct((n, 2), jnp.float32),
        jax.ShapeDtypeStruct((n, 2), jnp.int32),
        jax.ShapeDtypeStruct((num_e, 2, n), jnp.int32),
    )
    final, logits, weights, indices, mask = pl.pallas_call(
        _moe_kernel,
        grid=grid,
        in_specs=[
            pl.BlockSpec((blk, d), lambda i, e: (i, 0)),
            pl.BlockSpec((num_e, d), lambda i, e: (0, 0)),
            pl.BlockSpec((1, num_e), lambda i, e: (0, 0)),
            pl.BlockSpec((num_e, 1), lambda i, e: (0, 0)),
            pl.BlockSpec((1, d, d), lambda i, e: (e, 0, 0)),
            pl.BlockSpec((1, 1, d), lambda i, e: (e, 0, 0)),
        ],
        out_specs=(
            pl.BlockSpec((blk, d), lambda i, e: (i, 0)),
            pl.BlockSpec((blk, num_e), lambda i, e: (i, 0)),
            pl.BlockSpec((blk, 2), lambda i, e: (i, 0)),
            pl.BlockSpec((blk, 2), lambda i, e: (i, 0)),
            pl.BlockSpec((num_e, 2, blk), lambda i, e: (0, 0, i)),
        ),
        out_shape=out_shapes,
        scratch_shapes=[
            pltpu.VMEM((blk, 2), jnp.float32),
            pltpu.VMEM((blk, 2), jnp.int32),
        ],
    )(h, Wg, bg.reshape(1, num_e), bg.reshape(num_e, 1), We,
      be.reshape(num_e, 1, d))

    return (final.reshape(b, s, d), logits, weights, indices, mask)


# speedup vs baseline: 1.3990x; 1.0141x over previous
import functools

import jax
import jax.numpy as jnp
from jax.experimental import pallas as pl
from jax.experimental.pallas import tpu as pltpu


def _moe_kernel(x_ref, wg_ref, bg_row_ref, bg_col_ref, we_ref, be_ref,
                out_ref, logits_ref, w_ref, idx_ref, mask_ref,
                wscr, iscr, xbf):
    e = pl.program_id(1)
    num_e = pl.num_programs(1)

    @pl.when(e == 0)
    def _router():
        x = x_ref[...]
        xbf[...] = x.astype(jnp.bfloat16)
        wg = wg_ref[...]
        logits = jax.lax.dot_general(
            x, wg, (((1,), (1,)), ((), ())),
            preferred_element_type=jnp.float32) + bg_row_ref[...]
        logits_ref[...] = logits
        mx = jnp.max(logits, axis=1, keepdims=True)
        ex = jnp.exp(logits - mx)
        probs = ex / jnp.sum(ex, axis=1, keepdims=True)
        iota_e = jax.lax.broadcasted_iota(jnp.int32, probs.shape, 1)
        p0 = jnp.max(probs, axis=1, keepdims=True)
        i0 = jnp.min(jnp.where(probs == p0, iota_e, num_e),
                     axis=1, keepdims=True)
        probs1 = jnp.where(iota_e == i0, -1.0, probs)
        p1 = jnp.max(probs1, axis=1, keepdims=True)
        i1 = jnp.min(jnp.where(probs1 == p1, iota_e, num_e),
                     axis=1, keepdims=True)
        s = p0 + p1
        w0 = p0 / s
        w1 = p1 / s
        w_ref[:, 0:1] = w0
        w_ref[:, 1:2] = w1
        idx_ref[:, 0:1] = i0
        idx_ref[:, 1:2] = i1
        wscr[:, 0:1] = w0
        wscr[:, 1:2] = w1
        iscr[:, 0:1] = i0
        iscr[:, 1:2] = i1
        logits_t = jax.lax.dot_general(
            wg, x, (((1,), (1,)), ((), ())),
            preferred_element_type=jnp.float32) + bg_col_ref[...]
        mx_t = jnp.max(logits_t, axis=0, keepdims=True)
        ex_t = jnp.exp(logits_t - mx_t)
        probs_t = ex_t / jnp.sum(ex_t, axis=0, keepdims=True)
        iota_t = jax.lax.broadcasted_iota(jnp.int32, probs_t.shape, 0)
        p0_t = jnp.max(probs_t, axis=0, keepdims=True)
        i0_t = jnp.min(jnp.where(probs_t == p0_t, iota_t, num_e),
                       axis=0, keepdims=True)
        probs1_t = jnp.where(iota_t == i0_t, -1.0, probs_t)
        p1_t = jnp.max(probs1_t, axis=0, keepdims=True)
        i1_t = jnp.min(jnp.where(probs1_t == p1_t, iota_t, num_e),
                       axis=0, keepdims=True)
        mask_ref[:, 0, :] = (iota_t == i0_t).astype(jnp.int32)
        mask_ref[:, 1, :] = (iota_t == i1_t).astype(jnp.int32)

    w0 = wscr[:, 0:1]
    w1 = wscr[:, 1:2]
    i0 = iscr[:, 0:1]
    i1 = iscr[:, 1:2]
    we = we_ref[0]
    eo = jax.lax.dot_general(
        xbf[...], we, (((1,), (1,)), ((), ())),
        preferred_element_type=jnp.float32) + be_ref[0]
    w_e = jnp.where(i0 == e, w0, 0.0) + jnp.where(i1 == e, w1, 0.0)
    contrib = eo * w_e

    @pl.when(e == 0)
    def _init():
        out_ref[...] = contrib

    @pl.when(e > 0)
    def _acc():
        out_ref[...] = out_ref[...] + contrib


def kernel(x, Wg, bg, We, be):
    b, s, d = x.shape
    n = b * s
    num_e = Wg.shape[0]
    h = x.reshape(n, d)
    blk = 1024 if n % 1024 == 0 else n
    grid = (n // blk, num_e)

    out_shapes = (
        jax.ShapeDtypeStruct((n, d), jnp.float32),
        jax.ShapeDtypeStruct((n, num_e), jnp.float32),
        jax.ShapeDtypeStruct((n, 2), jnp.float32),
        jax.ShapeDtypeStruct((n, 2), jnp.int32),
        jax.ShapeDtypeStruct((num_e, 2, n), jnp.int32),
    )
    final, logits, weights, indices, mask = pl.pallas_call(
        _moe_kernel,
        grid=grid,
        in_specs=[
            pl.BlockSpec((blk, d), lambda i, e: (i, 0)),
            pl.BlockSpec((num_e, d), lambda i, e: (0, 0)),
            pl.BlockSpec((1, num_e), lambda i, e: (0, 0)),
            pl.BlockSpec((num_e, 1), lambda i, e: (0, 0)),
            pl.BlockSpec((1, d, d), lambda i, e: (e, 0, 0)),
            pl.BlockSpec((1, 1, d), lambda i, e: (e, 0, 0)),
        ],
        out_specs=(
            pl.BlockSpec((blk, d), lambda i, e: (i, 0)),
            pl.BlockSpec((blk, num_e), lambda i, e: (i, 0)),
            pl.BlockSpec((blk, 2), lambda i, e: (i, 0)),
            pl.BlockSpec((blk, 2), lambda i, e: (i, 0)),
            pl.BlockSpec((num_e, 2, blk), lambda i, e: (0, 0, i)),
        ),
        out_shape=out_shapes,
        scratch_shapes=[
            pltpu.VMEM((blk, 2), jnp.float32),
            pltpu.VMEM((blk, 2), jnp.int32),
            pltpu.VMEM((blk, d), jnp.bfloat16),
        ],
    )(h, Wg, bg.reshape(1, num_e), bg.reshape(num_e, 1),
      We.astype(jnp.bfloat16), be.reshape(num_e, 1, d))

    return (final.reshape(b, s, d), logits, weights, indices, mask)
